# baseline (speedup 1.0000x reference)
import functools
import numpy as np
import jax
import jax.numpy as jnp
from jax import lax
from jax.experimental import pallas as pl
from jax.experimental.pallas import tpu as pltpu
from jax.experimental.pallas import tpu_sc as plsc

_NPG = 57
_NBR = 80
_H = 128
_L = 5
_GB = 8
_R = _NPG * _GB
_BRANCH_U = np.arange(80) % 57
_BRANCH_V = (5 * np.arange(80) + 3) % 57

_INTERPRET = False


def _mm_kernel(h_ref, w_ref, b_ref, o_ref):
    o_ref[...] = h_ref[...] @ w_ref[...] + b_ref[...]


def _mm_scale_kernel(h_ref, w_ref, d_ref, o_ref):
    dinv = jax.lax.rsqrt(d_ref[...] + 1.0)
    o_ref[...] = (h_ref[...] @ w_ref[...]) * dinv


def _post_scatter_kernel(S_ref, g_ref, d_ref, bg_ref, s1_ref, s2_ref,
                         *, n_true, rows):
    dinv = jax.lax.rsqrt(d_ref[...] + 1.0)
    a = dinv * (S_ref[...] + g_ref[...]) + bg_ref[...]
    i = pl.program_id(0)
    row = i * rows + jax.lax.broadcasted_iota(jnp.int32, (rows, 1), 0)
    am = jnp.where(row < n_true, a, 0.0)

    @pl.when(i == 0)
    def _():
        s1_ref[...] = jnp.zeros_like(s1_ref)
        s2_ref[...] = jnp.zeros_like(s2_ref)

    s1_ref[...] += jnp.sum(am, axis=0, keepdims=True)
    s2_ref[...] += jnp.sum(am * am, axis=0, keepdims=True)


def _bn_relu_res_kernel(S_ref, g_ref, d_ref, bg_ref, res_ref, s1_ref,
                        s2_ref, gm_ref, bt_ref, o_ref, *, n_true):
    dinv = jax.lax.rsqrt(d_ref[...] + 1.0)
    a = dinv * (S_ref[...] + g_ref[...]) + bg_ref[...]
    mean = s1_ref[...] / n_true
    var = s2_ref[...] / n_true - mean * mean
    rstd = jax.lax.rsqrt(var + 1e-5)
    h = (a - mean) * rstd * gm_ref[...] + bt_ref[...]
    o_ref[...] = jnp.maximum(h, 0.0) + res_ref[...]


def _bn_relu_res_mm_kernel(S_ref, gin_ref, d_ref, bg_ref, res_ref, s1_ref,
                           s2_ref, gm_ref, bt_ref, w_ref, h_ref, g_ref,
                           *, n_true):
    dinv = jax.lax.rsqrt(d_ref[...] + 1.0)
    a = dinv * (S_ref[...] + gin_ref[...]) + bg_ref[...]
    mean = s1_ref[...] / n_true
    var = s2_ref[...] / n_true - mean * mean
    rstd = jax.lax.rsqrt(var + 1e-5)
    h = (a - mean) * rstd * gm_ref[...] + bt_ref[...]
    h = jnp.maximum(h, 0.0) + res_ref[...]
    h_ref[...] = h
    g_ref[...] = (h @ w_ref[...]) * dinv


def _final_kernel(h_ref, bu_ref, bv_ref, bd_ref, wp1_ref, bp1_ref, wp2_ref,
                  bp2_ref, wa_ref, wb_ref, wc_ref, bm1_ref, wm2_ref, bm2_ref,
                  o_ref):
    hb = h_ref[...]
    t = jnp.maximum(hb @ wp1_ref[...] + bp1_ref[...], 0.0)
    t = t @ wp2_ref[...] + bp2_ref[...]
    hu = bu_ref[...] @ hb
    hv = bv_ref[...] @ hb
    dth = bd_ref[...] @ t
    hid = hu @ wa_ref[...] + hv @ wb_ref[...] + dth @ wc_ref[...] + bm1_ref[...]
    hid = jnp.maximum(hid, 0.0)
    o_ref[...] = hid @ wm2_ref[...] + bm2_ref[...]


def _row_block_call(body, grid, out_shapes, row_specs, full_specs, interpret):
    return pl.pallas_call(
        body,
        grid=grid,
        in_specs=row_specs + full_specs,
        out_shape=out_shapes[0] if len(out_shapes) == 1 else out_shapes,
        out_specs=None,
        interpret=interpret,
    )


_BD = 352
_ACC = 368
_EB = 128
_NW = 32
_ZR = 184


def _sc_prep(src, dst, n_pad):
    e = dst.shape[0]
    e_pad = ((e + _EB - 1) // _EB) * _EB
    nblk = n_pad // _BD
    perm = jnp.argsort(dst)
    dst_s = jnp.pad(dst[perm], (0, e_pad - e), constant_values=n_pad)
    src_s = jnp.pad(src[perm], (0, e_pad - e))
    ldst = jnp.where(dst_s < n_pad, dst_s % _BD, 0).astype(jnp.int32)
    edges = jnp.arange(nblk + 1, dtype=jnp.int32) * _BD
    bptr = jnp.searchsorted(dst_s, edges, side="left").astype(jnp.int32)
    bptr = jnp.pad(bptr, (0, 304 - (nblk + 1)))
    return src_s.astype(jnp.int32), ldst, bptr, nblk, e_pad


def _sc_scatter(g, src_s, ldst, bptr, n_pad, nblk):
    mesh = plsc.VectorSubcoreMesh(core_axis_name="c", subcore_axis_name="s")
    nt = (nblk + _NW - 1) // _NW

    @functools.partial(
        pl.kernel, mesh=mesh,
        out_type=jax.ShapeDtypeStruct((n_pad, _H), jnp.float32),
        scratch_types=[
            pltpu.VMEM((304,), jnp.int32),
            pltpu.VMEM((_EB,), jnp.int32),
            pltpu.VMEM((_EB,), jnp.int32),
            pltpu.VMEM((_EB,), jnp.int32),
            pltpu.VMEM((_EB,), jnp.int32),
            pltpu.VMEM((_EB, _H), jnp.float32),
            pltpu.VMEM((_EB, _H), jnp.float32),
            pltpu.VMEM((_ZR, _H), jnp.float32),
            pltpu.VMEM_SHARED((16 * _ACC, _H), jnp.float32),
            pltpu.SemaphoreType.DMA,
            pltpu.SemaphoreType.DMA,
        ],
    )
    def sc_fn(g_hbm, src_hbm, ldst_hbm, bptr_hbm, s_hbm,
              bptr_v, src_v0, src_v1, ldst_v0, ldst_v1, rows_v0, rows_v1,
              zbuf, acc_sh, sem0, sem1):
        cid = lax.axis_index("c")
        sid = lax.axis_index("s")
        wid = cid * 16 + sid
        w_off = sid * _ACC
        dummy = w_off + _BD
        bufs = [(src_v0, ldst_v0, rows_v0, sem0),
                (src_v1, ldst_v1, rows_v1, sem1)]

        pltpu.sync_copy(bptr_hbm, bptr_v)

        def zero_row(i, _):
            for j in range(_H // 16):
                zbuf[i, pl.ds(j * 16, 16)] = jnp.zeros((16,), jnp.float32)
            return 0

        lax.fori_loop(0, _ZR, zero_row, 0)

        for t in range(nt):
            k = wid + _NW * t

            @pl.when(k < nblk)
            def _():
                bv = bptr_v[pl.ds(k, 16)]
                e_lo = bv[0]
                e_hi = bv[1]
                base0 = (e_lo // _EB) * _EB
                nb = (e_hi - base0 + _EB - 1) // _EB

                for r in range(_ACC // _ZR):
                    pltpu.sync_copy(
                        zbuf, acc_sh.at[pl.ds(w_off + r * _ZR, _ZR)])

                def issue(bi, buf):
                    src_v, ldst_v, rows_v, sem = buf
                    b0 = base0 + bi * _EB
                    pltpu.sync_copy(src_hbm.at[pl.ds(b0, _EB)], src_v)
                    pltpu.sync_copy(ldst_hbm.at[pl.ds(b0, _EB)], ldst_v)
                    for j in range(_EB // 16):
                        gid = b0 + j * 16 + lax.iota(jnp.int32, 16)
                        seg = ldst_v[pl.ds(j * 16, 16)]
                        ok = (gid >= e_lo) & (gid < e_hi)
                        ldst_v[pl.ds(j * 16, 16)] = jnp.where(
                            ok, seg + w_off, dummy)
                    pltpu.async_copy(g_hbm.at[src_v], rows_v, sem)

                def drain(buf):
                    src_v, ldst_v, rows_v, sem = buf
                    pltpu.make_async_copy(
                        g_hbm.at[src_v], rows_v, sem).wait()
                    pltpu.sync_copy(rows_v, acc_sh.at[ldst_v], add=True)

                @pl.when(nb > 0)
                def _():
                    issue(0, bufs[0])

                def batch_body(bi, _):
                    @pl.when(bi % 2 == 0)
                    def _():
                        issue(bi + 1, bufs[1])
                        drain(bufs[0])

                    @pl.when(bi % 2 == 1)
                    def _():
                        issue(bi + 1, bufs[0])
                        drain(bufs[1])

                    return 0

                lax.fori_loop(0, nb - 1, batch_body, 0)

                @pl.when(nb > 0)
                def _():
                    @pl.when((nb - 1) % 2 == 0)
                    def _():
                        drain(bufs[0])

                    @pl.when((nb - 1) % 2 == 1)
                    def _():
                        drain(bufs[1])

                pltpu.sync_copy(acc_sh.at[pl.ds(w_off, _BD)],
                                s_hbm.at[pl.ds(k * _BD, _BD)])

    return sc_fn(g, src_s, ldst, bptr)


def _sc_degree(ldst, bptr, n_pad, nblk):
    mesh = plsc.VectorSubcoreMesh(core_axis_name="c", subcore_axis_name="s")
    nt = (nblk + _NW - 1) // _NW

    @functools.partial(
        pl.kernel, mesh=mesh,
        out_type=jax.ShapeDtypeStruct((n_pad, _H), jnp.float32),
        scratch_types=[
            pltpu.VMEM((304,), jnp.int32),
            pltpu.VMEM((_EB,), jnp.int32),
            pltpu.VMEM((_EB, _H), jnp.float32),
            pltpu.VMEM((_ZR, _H), jnp.float32),
            pltpu.VMEM_SHARED((16 * _ACC, _H), jnp.float32),
            pltpu.SemaphoreType.DMA,
        ],
    )
    def sc_fn(ldst_hbm, bptr_hbm, deg_hbm, bptr_v, ldst_v, ones_v, zbuf,
              acc_sh, sem):
        cid = lax.axis_index("c")
        sid = lax.axis_index("s")
        wid = cid * 16 + sid
        w_off = sid * _ACC
        dummy = w_off + _BD

        pltpu.sync_copy(bptr_hbm, bptr_v)

        def fill_row(i, _):
            for j in range(_H // 16):
                ones_v[i, pl.ds(j * 16, 16)] = jnp.ones((16,), jnp.float32)
            return 0

        lax.fori_loop(0, _EB, fill_row, 0)

        def zero_row(i, _):
            for j in range(_H // 16):
                zbuf[i, pl.ds(j * 16, 16)] = jnp.zeros((16,), jnp.float32)
            return 0

        lax.fori_loop(0, _ZR, zero_row, 0)

        for t in range(nt):
            k = wid + _NW * t

            @pl.when(k < nblk)
            def _():
                bv = bptr_v[pl.ds(k, 16)]
                e_lo = bv[0]
                e_hi = bv[1]
                base0 = (e_lo // _EB) * _EB
                nb = (e_hi - base0 + _EB - 1) // _EB

                for r in range(_ACC // _ZR):
                    pltpu.sync_copy(
                        zbuf, acc_sh.at[pl.ds(w_off + r * _ZR, _ZR)])

                def batch_body(bi, _):
                    b0 = base0 + bi * _EB
                    pltpu.sync_copy(ldst_hbm.at[pl.ds(b0, _EB)], ldst_v)
                    for j in range(_EB // 16):
                        gid = b0 + j * 16 + lax.iota(jnp.int32, 16)
                        seg = ldst_v[pl.ds(j * 16, 16)]
                        ok = (gid >= e_lo) & (gid < e_hi)
                        ldst_v[pl.ds(j * 16, 16)] = jnp.where(
                            ok, seg + w_off, dummy)
                    pltpu.sync_copy(ones_v, acc_sh.at[ldst_v], add=True)
                    return 0

                lax.fori_loop(0, nb, batch_body, 0)
                pltpu.sync_copy(acc_sh.at[pl.ds(w_off, _BD)],
                                deg_hbm.at[pl.ds(k * _BD, _BD)])

    return sc_fn(ldst, bptr)


def kernel(x, edge_index, num_graphs, W0, b0, Wg, bg, gamma, beta,
           Wp1, bp1, Wp2, bp2, Wm1, bm1, Wm2, bm2):
    n = x.shape[0]
    G = n // _NPG
    G_pad = ((G + _GB - 1) // _GB) * _GB
    n_pad = G_pad * _NPG
    steps = n_pad // _R
    itp = _INTERPRET

    src = edge_index[0]
    dst = edge_index[1]
    src_s, ldst, bptr, nblk, _ = _sc_prep(src, dst, n_pad)
    deg = _sc_degree(ldst, bptr, n_pad, nblk)[:, :1]

    x_pad = jnp.pad(x, ((0, n_pad - n), (0, 0)))

    row_spec = pl.BlockSpec((_R, _H), lambda i: (i, 0))
    col_spec = pl.BlockSpec((_R, 1), lambda i: (i, 0))
    stat_spec = pl.BlockSpec((1, _H), lambda i: (0, 0))

    def full(shape):
        return pl.BlockSpec(shape, lambda i: tuple(0 for _ in shape))

    h = pl.pallas_call(
        _mm_kernel,
        grid=(steps,),
        in_specs=[row_spec, full((_H, _H)), full((1, _H))],
        out_specs=row_spec,
        out_shape=jax.ShapeDtypeStruct((n_pad, _H), jnp.float32),
        interpret=itp,
    )(x_pad, W0, b0[None, :])

    g = pl.pallas_call(
        _mm_scale_kernel,
        grid=(steps,),
        in_specs=[row_spec, full((_H, _H)), col_spec],
        out_specs=row_spec,
        out_shape=jax.ShapeDtypeStruct((n_pad, _H), jnp.float32),
        interpret=itp,
    )(h, Wg[0], deg)

    for i in range(_L):
        res = h
        S = _sc_scatter(g, src_s, ldst, bptr, n_pad, nblk)

        s1, s2 = pl.pallas_call(
            functools.partial(_post_scatter_kernel, n_true=n, rows=_R),
            grid=(steps,),
            in_specs=[row_spec, row_spec, col_spec, full((1, _H))],
            out_specs=[stat_spec, stat_spec],
            out_shape=[
                jax.ShapeDtypeStruct((1, _H), jnp.float32),
                jax.ShapeDtypeStruct((1, _H), jnp.float32),
            ],
            interpret=itp,
        )(S, g, deg, bg[i][None, :])

        if i < _L - 1:
            h, g = pl.pallas_call(
                functools.partial(_bn_relu_res_mm_kernel, n_true=float(n)),
                grid=(steps,),
                in_specs=[row_spec, row_spec, col_spec, full((1, _H)),
                          row_spec, stat_spec, stat_spec,
                          full((1, _H)), full((1, _H)), full((_H, _H))],
                out_specs=[row_spec, row_spec],
                out_shape=[
                    jax.ShapeDtypeStruct((n_pad, _H), jnp.float32),
                    jax.ShapeDtypeStruct((n_pad, _H), jnp.float32),
                ],
                interpret=itp,
            )(S, g, deg, bg[i][None, :], res, s1, s2,
              gamma[i][None, :], beta[i][None, :], Wg[i + 1])
        else:
            h = pl.pallas_call(
                functools.partial(_bn_relu_res_kernel, n_true=float(n)),
                grid=(steps,),
                in_specs=[row_spec, row_spec, col_spec, full((1, _H)),
                          row_spec, stat_spec, stat_spec,
                          full((1, _H)), full((1, _H))],
                out_specs=row_spec,
                out_shape=jax.ShapeDtypeStruct((n_pad, _H), jnp.float32),
                interpret=itp,
            )(S, g, deg, bg[i][None, :], res, s1, s2,
              gamma[i][None, :], beta[i][None, :])

    U1 = np.zeros((_NBR, _NPG), np.float32)
    U1[np.arange(_NBR), _BRANCH_U] = 1.0
    V1 = np.zeros((_NBR, _NPG), np.float32)
    V1[np.arange(_NBR), _BRANCH_V] = 1.0
    BU = jnp.asarray(np.kron(np.eye(_GB, dtype=np.float32), U1))
    BV = jnp.asarray(np.kron(np.eye(_GB, dtype=np.float32), V1))
    BD = BU - BV
    EB = _GB * _NBR

    out = pl.pallas_call(
        _final_kernel,
        grid=(G_pad // _GB,),
        in_specs=[row_spec,
                  full((EB, _R)), full((EB, _R)), full((EB, _R)),
                  full((_H, 16)), full((1, 16)), full((16, 1)), full((1, 1)),
                  full((_H, _H)), full((_H, _H)), full((1, _H)),
                  full((1, _H)), full((_H, 1)), full((1, 1))],
        out_specs=pl.BlockSpec((EB, 1), lambda i: (i, 0)),
        out_shape=jax.ShapeDtypeStruct((G_pad * _NBR, 1), jnp.float32),
        interpret=itp,
    )(h, BU, BV, BD,
      Wp1, bp1[None, :], Wp2, bp2[None, :],
      Wm1[:_H], Wm1[_H:2 * _H], Wm1[2 * _H:2 * _H + 1],
      bm1[None, :], Wm2, bm2[None, :])

    return out[:G * _NBR]


# speedup vs baseline: 1.0133x; 1.0133x over previous
import functools
import numpy as np
import jax
import jax.numpy as jnp
from jax import lax
from jax.experimental import pallas as pl
from jax.experimental.pallas import tpu as pltpu
from jax.experimental.pallas import tpu_sc as plsc

_NPG = 57
_NBR = 80
_H = 128
_L = 5
_GB = 8
_R = _NPG * _GB
_BRANCH_U = np.arange(80) % 57
_BRANCH_V = (5 * np.arange(80) + 3) % 57


def _mm_kernel(h_ref, w_ref, b_ref, o_ref):
    o_ref[...] = h_ref[...] @ w_ref[...] + b_ref[...]


def _mm_scale_kernel(h_ref, w_ref, d_ref, o_ref):
    dinv = jax.lax.rsqrt(d_ref[...] + 1.0)
    o_ref[...] = (h_ref[...] @ w_ref[...]) * dinv


def _post_scatter_kernel(S_ref, g_ref, d_ref, bg_ref, o_ref, s1_ref, s2_ref,
                         *, n_true, rows):
    dinv = jax.lax.rsqrt(d_ref[...] + 1.0)
    a = dinv * (S_ref[...] + g_ref[...]) + bg_ref[...]
    o_ref[...] = a
    i = pl.program_id(0)
    row = i * rows + jax.lax.broadcasted_iota(jnp.int32, (rows, 1), 0)
    am = jnp.where(row < n_true, a, 0.0)

    @pl.when(i == 0)
    def _():
        s1_ref[...] = jnp.zeros_like(s1_ref)
        s2_ref[...] = jnp.zeros_like(s2_ref)

    s1_ref[...] += jnp.sum(am, axis=0, keepdims=True)
    s2_ref[...] += jnp.sum(am * am, axis=0, keepdims=True)


def _bn_relu_res_kernel(a_ref, res_ref, s1_ref, s2_ref, gm_ref, bt_ref,
                        o_ref, *, n_true):
    a = a_ref[...]
    mean = s1_ref[...] / n_true
    var = s2_ref[...] / n_true - mean * mean
    rstd = jax.lax.rsqrt(var + 1e-5)
    h = (a - mean) * rstd * gm_ref[...] + bt_ref[...]
    o_ref[...] = jnp.maximum(h, 0.0) + res_ref[...]


def _bn_relu_res_mm_kernel(a_ref, res_ref, s1_ref, s2_ref, gm_ref, bt_ref,
                           w_ref, d_ref, h_ref, g_ref, *, n_true):
    a = a_ref[...]
    mean = s1_ref[...] / n_true
    var = s2_ref[...] / n_true - mean * mean
    rstd = jax.lax.rsqrt(var + 1e-5)
    h = (a - mean) * rstd * gm_ref[...] + bt_ref[...]
    h = jnp.maximum(h, 0.0) + res_ref[...]
    h_ref[...] = h
    dinv = jax.lax.rsqrt(d_ref[...] + 1.0)
    g_ref[...] = (h @ w_ref[...]) * dinv


def _final_kernel(h_ref, bu_ref, bv_ref, bd_ref, wp1_ref, bp1_ref, wp2_ref,
                  bp2_ref, wa_ref, wb_ref, wc_ref, bm1_ref, wm2_ref, bm2_ref,
                  o_ref):
    hb = h_ref[...]
    t = jnp.maximum(hb @ wp1_ref[...] + bp1_ref[...], 0.0)
    t = t @ wp2_ref[...] + bp2_ref[...]
    hu = bu_ref[...] @ hb
    hv = bv_ref[...] @ hb
    dth = bd_ref[...] @ t
    hid = hu @ wa_ref[...] + hv @ wb_ref[...] + dth @ wc_ref[...] + bm1_ref[...]
    hid = jnp.maximum(hid, 0.0)
    o_ref[...] = hid @ wm2_ref[...] + bm2_ref[...]


_BD = 352
_ACC = 368
_EB = 128
_NW = 32
_ZR = 184


def _sc_prep(src, dst, n_pad):
    e = dst.shape[0]
    e_pad = ((e + _EB - 1) // _EB) * _EB
    nblk = n_pad // _BD
    perm = jnp.argsort(dst)
    dst_s = jnp.pad(dst[perm], (0, e_pad - e), constant_values=n_pad)
    src_s = jnp.pad(src[perm], (0, e_pad - e))
    ldst = jnp.where(dst_s < n_pad, dst_s % _BD, 0).astype(jnp.int32)
    edges = jnp.arange(nblk + 1, dtype=jnp.int32) * _BD
    bptr = jnp.searchsorted(dst_s, edges, side="left").astype(jnp.int32)
    bptr = jnp.pad(bptr, (0, 304 - (nblk + 1)))
    return src_s.astype(jnp.int32), ldst, bptr, nblk, e_pad


def _sc_scatter(g, src_s, ldst, bptr, n_pad, nblk):
    mesh = plsc.VectorSubcoreMesh(core_axis_name="c", subcore_axis_name="s")
    nt = (nblk + _NW - 1) // _NW

    @functools.partial(
        pl.kernel, mesh=mesh,
        out_type=jax.ShapeDtypeStruct((n_pad, _H), jnp.float32),
        scratch_types=[
            pltpu.VMEM((304,), jnp.int32),
            pltpu.VMEM((_EB,), jnp.int32),
            pltpu.VMEM((_EB,), jnp.int32),
            pltpu.VMEM((_EB,), jnp.int32),
            pltpu.VMEM((_EB,), jnp.int32),
            pltpu.VMEM((_EB, _H), jnp.float32),
            pltpu.VMEM((_EB, _H), jnp.float32),
            pltpu.VMEM((_ZR, _H), jnp.float32),
            pltpu.VMEM_SHARED((16 * _ACC, _H), jnp.float32),
            pltpu.SemaphoreType.DMA,
            pltpu.SemaphoreType.DMA,
        ],
    )
    def sc_fn(g_hbm, src_hbm, ldst_hbm, bptr_hbm, s_hbm,
              bptr_v, src_v0, src_v1, ldst_v0, ldst_v1, rows_v0, rows_v1,
              zbuf, acc_sh, sem0, sem1):
        cid = lax.axis_index("c")
        sid = lax.axis_index("s")
        wid = cid * 16 + sid
        w_off = sid * _ACC
        dummy = w_off + _BD
        bufs = [(src_v0, ldst_v0, rows_v0, sem0),
                (src_v1, ldst_v1, rows_v1, sem1)]

        pltpu.sync_copy(bptr_hbm, bptr_v)

        def zero_row(i, _):
            for j in range(_H // 16):
                zbuf[i, pl.ds(j * 16, 16)] = jnp.zeros((16,), jnp.float32)
            return 0

        lax.fori_loop(0, _ZR, zero_row, 0)

        for t in range(nt):
            k = wid + _NW * t

            @pl.when(k < nblk)
            def _():
                bv = bptr_v[pl.ds(k, 16)]
                e_lo = bv[0]
                e_hi = bv[1]
                base0 = (e_lo // _EB) * _EB
                nb = (e_hi - base0 + _EB - 1) // _EB

                for r in range(_ACC // _ZR):
                    pltpu.sync_copy(
                        zbuf, acc_sh.at[pl.ds(w_off + r * _ZR, _ZR)])

                def issue(bi, buf):
                    src_v, ldst_v, rows_v, sem = buf
                    b0 = base0 + bi * _EB
                    pltpu.sync_copy(src_hbm.at[pl.ds(b0, _EB)], src_v)
                    pltpu.sync_copy(ldst_hbm.at[pl.ds(b0, _EB)], ldst_v)
                    for j in range(_EB // 16):
                        gid = b0 + j * 16 + lax.iota(jnp.int32, 16)
                        seg = ldst_v[pl.ds(j * 16, 16)]
                        ok = (gid >= e_lo) & (gid < e_hi)
                        ldst_v[pl.ds(j * 16, 16)] = jnp.where(
                            ok, seg + w_off, dummy)
                    pltpu.async_copy(g_hbm.at[src_v], rows_v, sem)

                def drain(buf):
                    src_v, ldst_v, rows_v, sem = buf
                    pltpu.make_async_copy(
                        g_hbm.at[src_v], rows_v, sem).wait()
                    pltpu.sync_copy(rows_v, acc_sh.at[ldst_v], add=True)

                @pl.when(nb > 0)
                def _():
                    issue(0, bufs[0])

                def batch_body(bi, _):
                    @pl.when(bi % 2 == 0)
                    def _():
                        issue(bi + 1, bufs[1])
                        drain(bufs[0])

                    @pl.when(bi % 2 == 1)
                    def _():
                        issue(bi + 1, bufs[0])
                        drain(bufs[1])

                    return 0

                lax.fori_loop(0, nb - 1, batch_body, 0)

                @pl.when(nb > 0)
                def _():
                    @pl.when((nb - 1) % 2 == 0)
                    def _():
                        drain(bufs[0])

                    @pl.when((nb - 1) % 2 == 1)
                    def _():
                        drain(bufs[1])

                pltpu.sync_copy(acc_sh.at[pl.ds(w_off, _BD)],
                                s_hbm.at[pl.ds(k * _BD, _BD)])

    return sc_fn(g, src_s, ldst, bptr)


def _sc_degree(ldst, bptr, n_pad, nblk):
    mesh = plsc.VectorSubcoreMesh(core_axis_name="c", subcore_axis_name="s")
    nt = (nblk + _NW - 1) // _NW

    @functools.partial(
        pl.kernel, mesh=mesh,
        out_type=jax.ShapeDtypeStruct((n_pad, _H), jnp.float32),
        scratch_types=[
            pltpu.VMEM((304,), jnp.int32),
            pltpu.VMEM((_EB,), jnp.int32),
            pltpu.VMEM((_EB, _H), jnp.float32),
            pltpu.VMEM((_ZR, _H), jnp.float32),
            pltpu.VMEM_SHARED((16 * _ACC, _H), jnp.float32),
            pltpu.SemaphoreType.DMA,
        ],
    )
    def sc_fn(ldst_hbm, bptr_hbm, deg_hbm, bptr_v, ldst_v, ones_v, zbuf,
              acc_sh, sem):
        cid = lax.axis_index("c")
        sid = lax.axis_index("s")
        wid = cid * 16 + sid
        w_off = sid * _ACC
        dummy = w_off + _BD

        pltpu.sync_copy(bptr_hbm, bptr_v)

        def fill_row(i, _):
            for j in range(_H // 16):
                ones_v[i, pl.ds(j * 16, 16)] = jnp.ones((16,), jnp.float32)
            return 0

        lax.fori_loop(0, _EB, fill_row, 0)

        def zero_row(i, _):
            for j in range(_H // 16):
                zbuf[i, pl.ds(j * 16, 16)] = jnp.zeros((16,), jnp.float32)
            return 0

        lax.fori_loop(0, _ZR, zero_row, 0)

        for t in range(nt):
            k = wid + _NW * t

            @pl.when(k < nblk)
            def _():
                bv = bptr_v[pl.ds(k, 16)]
                e_lo = bv[0]
                e_hi = bv[1]
                base0 = (e_lo // _EB) * _EB
                nb = (e_hi - base0 + _EB - 1) // _EB

                for r in range(_ACC // _ZR):
                    pltpu.sync_copy(
                        zbuf, acc_sh.at[pl.ds(w_off + r * _ZR, _ZR)])

                def batch_body(bi, _):
                    b0 = base0 + bi * _EB
                    pltpu.sync_copy(ldst_hbm.at[pl.ds(b0, _EB)], ldst_v)
                    for j in range(_EB // 16):
                        gid = b0 + j * 16 + lax.iota(jnp.int32, 16)
                        seg = ldst_v[pl.ds(j * 16, 16)]
                        ok = (gid >= e_lo) & (gid < e_hi)
                        ldst_v[pl.ds(j * 16, 16)] = jnp.where(
                            ok, seg + w_off, dummy)
                    pltpu.sync_copy(ones_v, acc_sh.at[ldst_v], add=True)
                    return 0

                lax.fori_loop(0, nb, batch_body, 0)
                pltpu.sync_copy(acc_sh.at[pl.ds(w_off, _BD)],
                                deg_hbm.at[pl.ds(k * _BD, _BD)])

    return sc_fn(ldst, bptr)


def kernel(x, edge_index, num_graphs, W0, b0, Wg, bg, gamma, beta,
           Wp1, bp1, Wp2, bp2, Wm1, bm1, Wm2, bm2):
    n = x.shape[0]
    G = n // _NPG
    G_pad = ((G + _GB - 1) // _GB) * _GB
    n_pad = G_pad * _NPG
    steps = n_pad // _R
    itp = False

    src = edge_index[0]
    dst = edge_index[1]
    src_s, ldst, bptr, nblk, _ = _sc_prep(src, dst, n_pad)
    deg = _sc_degree(ldst, bptr, n_pad, nblk)[:, :1]

    x_pad = jnp.pad(x, ((0, n_pad - n), (0, 0)))

    row_spec = pl.BlockSpec((_R, _H), lambda i: (i, 0))
    col_spec = pl.BlockSpec((_R, 1), lambda i: (i, 0))
    stat_spec = pl.BlockSpec((1, _H), lambda i: (0, 0))

    def full(shape):
        return pl.BlockSpec(shape, lambda i: tuple(0 for _ in shape))

    h = pl.pallas_call(
        _mm_kernel,
        grid=(steps,),
        in_specs=[row_spec, full((_H, _H)), full((1, _H))],
        out_specs=row_spec,
        out_shape=jax.ShapeDtypeStruct((n_pad, _H), jnp.float32),
        interpret=itp,
    )(x_pad, W0, b0[None, :])

    g = pl.pallas_call(
        _mm_scale_kernel,
        grid=(steps,),
        in_specs=[row_spec, full((_H, _H)), col_spec],
        out_specs=row_spec,
        out_shape=jax.ShapeDtypeStruct((n_pad, _H), jnp.float32),
        interpret=itp,
    )(h, Wg[0], deg)

    for i in range(_L):
        res = h
        S = _sc_scatter(g, src_s, ldst, bptr, n_pad, nblk)

        a, s1, s2 = pl.pallas_call(
            functools.partial(_post_scatter_kernel, n_true=n, rows=_R),
            grid=(steps,),
            in_specs=[row_spec, row_spec, col_spec, full((1, _H))],
            out_specs=[row_spec, stat_spec, stat_spec],
            out_shape=[
                jax.ShapeDtypeStruct((n_pad, _H), jnp.float32),
                jax.ShapeDtypeStruct((1, _H), jnp.float32),
                jax.ShapeDtypeStruct((1, _H), jnp.float32),
            ],
            interpret=itp,
        )(S, g, deg, bg[i][None, :])

        if i < _L - 1:
            h, g = pl.pallas_call(
                functools.partial(_bn_relu_res_mm_kernel, n_true=float(n)),
                grid=(steps,),
                in_specs=[row_spec, row_spec, stat_spec, stat_spec,
                          full((1, _H)), full((1, _H)), full((_H, _H)),
                          col_spec],
                out_specs=[row_spec, row_spec],
                out_shape=[
                    jax.ShapeDtypeStruct((n_pad, _H), jnp.float32),
                    jax.ShapeDtypeStruct((n_pad, _H), jnp.float32),
                ],
                interpret=itp,
            )(a, res, s1, s2, gamma[i][None, :], beta[i][None, :],
              Wg[i + 1], deg)
        else:
            h = pl.pallas_call(
                functools.partial(_bn_relu_res_kernel, n_true=float(n)),
                grid=(steps,),
                in_specs=[row_spec, row_spec, stat_spec, stat_spec,
                          full((1, _H)), full((1, _H))],
                out_specs=row_spec,
                out_shape=jax.ShapeDtypeStruct((n_pad, _H), jnp.float32),
                interpret=itp,
            )(a, res, s1, s2, gamma[i][None, :], beta[i][None, :])

    U1 = np.zeros((_NBR, _NPG), np.float32)
    U1[np.arange(_NBR), _BRANCH_U] = 1.0
    V1 = np.zeros((_NBR, _NPG), np.float32)
    V1[np.arange(_NBR), _BRANCH_V] = 1.0
    BU = jnp.asarray(np.kron(np.eye(_GB, dtype=np.float32), U1))
    BV = jnp.asarray(np.kron(np.eye(_GB, dtype=np.float32), V1))
    BD = BU - BV
    EB = _GB * _NBR

    out = pl.pallas_call(
        _final_kernel,
        grid=(G_pad // _GB,),
        in_specs=[row_spec,
                  full((EB, _R)), full((EB, _R)), full((EB, _R)),
                  full((_H, 16)), full((1, 16)), full((16, 1)), full((1, 1)),
                  full((_H, _H)), full((_H, _H)), full((1, _H)),
                  full((1, _H)), full((_H, 1)), full((1, 1))],
        out_specs=pl.BlockSpec((EB, 1), lambda i: (i, 0)),
        out_shape=jax.ShapeDtypeStruct((G_pad * _NBR, 1), jnp.float32),
        interpret=itp,
    )(h, BU, BV, BD,
      Wp1, bp1[None, :], Wp2, bp2[None, :],
      Wm1[:_H], Wm1[_H:2 * _H], Wm1[2 * _H:2 * _H + 1],
      bm1[None, :], Wm2, bm2[None, :])

    return out[:G * _NBR]


# speedup vs baseline: 1.0464x; 1.0326x over previous
import functools
import numpy as np
import jax
import jax.numpy as jnp
from jax import lax
from jax.experimental import pallas as pl
from jax.experimental.pallas import tpu as pltpu
from jax.experimental.pallas import tpu_sc as plsc

_NPG = 57
_NBR = 80
_H = 128
_L = 5
_GB = 8
_R = _NPG * _GB
_BRANCH_U = np.arange(80) % 57
_BRANCH_V = (5 * np.arange(80) + 3) % 57


def _mm_kernel(x_ref, w0_ref, b0_ref, wg_ref, d_ref, h_ref, g_ref):
    h = x_ref[...] @ w0_ref[...] + b0_ref[...]
    h_ref[...] = h
    dinv = jax.lax.rsqrt(d_ref[...] + 1.0)
    g_ref[...] = (h @ wg_ref[...]) * dinv


def _post_scatter_kernel(S_ref, g_ref, d_ref, bg_ref, o_ref, s1_ref, s2_ref,
                         *, n_true, rows):
    dinv = jax.lax.rsqrt(d_ref[...] + 1.0)
    a = dinv * (S_ref[...] + g_ref[...]) + bg_ref[...]
    o_ref[...] = a
    i = pl.program_id(0)
    row = i * rows + jax.lax.broadcasted_iota(jnp.int32, (rows, 1), 0)
    am = jnp.where(row < n_true, a, 0.0)

    @pl.when(i == 0)
    def _():
        s1_ref[...] = jnp.zeros_like(s1_ref)
        s2_ref[...] = jnp.zeros_like(s2_ref)

    s1_ref[...] += jnp.sum(am, axis=0, keepdims=True)
    s2_ref[...] += jnp.sum(am * am, axis=0, keepdims=True)


def _bn_relu_res_mm_kernel(a_ref, res_ref, s1_ref, s2_ref, gm_ref, bt_ref,
                           w_ref, d_ref, h_ref, g_ref, *, n_true):
    a = a_ref[...]
    mean = s1_ref[...] / n_true
    var = s2_ref[...] / n_true - mean * mean
    rstd = jax.lax.rsqrt(var + 1e-5)
    h = (a - mean) * rstd * gm_ref[...] + bt_ref[...]
    h = jnp.maximum(h, 0.0) + res_ref[...]
    h_ref[...] = h
    dinv = jax.lax.rsqrt(d_ref[...] + 1.0)
    g_ref[...] = (h @ w_ref[...]) * dinv


def _final_kernel(a_ref, res_ref, s1_ref, s2_ref, gm_ref, bt_ref,
                  bu_ref, bv_ref, bd_ref, wp1_ref, bp1_ref, wp2_ref,
                  bp2_ref, wa_ref, wb_ref, wc_ref, bm1_ref, wm2_ref, bm2_ref,
                  o_ref, *, n_true):
    mean = s1_ref[...] / n_true
    var = s2_ref[...] / n_true - mean * mean
    rstd = jax.lax.rsqrt(var + 1e-5)
    hb = (a_ref[...] - mean) * rstd * gm_ref[...] + bt_ref[...]
    hb = jnp.maximum(hb, 0.0) + res_ref[...]
    t = jnp.maximum(hb @ wp1_ref[...] + bp1_ref[...], 0.0)
    t = t @ wp2_ref[...] + bp2_ref[...]
    hu = bu_ref[...] @ hb
    hv = bv_ref[...] @ hb
    dth = bd_ref[...] @ t
    hid = hu @ wa_ref[...] + hv @ wb_ref[...] + dth @ wc_ref[...] + bm1_ref[...]
    hid = jnp.maximum(hid, 0.0)
    o_ref[...] = hid @ wm2_ref[...] + bm2_ref[...]


_BD = 352
_ACC = 368
_EB = 128
_NW = 32
_ZR = 184


def _sc_prep(src, dst, n_pad):
    e = dst.shape[0]
    e_pad = ((e + _EB - 1) // _EB) * _EB
    nblk = n_pad // _BD
    perm = jnp.argsort(dst)
    dst_s = jnp.pad(dst[perm], (0, e_pad - e), constant_values=n_pad)
    src_s = jnp.pad(src[perm], (0, e_pad - e))
    ldst = jnp.where(dst_s < n_pad, dst_s % _BD, 0).astype(jnp.int32)
    edges = jnp.arange(nblk + 1, dtype=jnp.int32) * _BD
    bptr = jnp.searchsorted(dst_s, edges, side="left").astype(jnp.int32)
    bptr = jnp.pad(bptr, (0, 304 - (nblk + 1)))
    return src_s.astype(jnp.int32), ldst, bptr, nblk, e_pad


def _sc_scatter(g, src_s, ldst, bptr, n_pad, nblk):
    mesh = plsc.VectorSubcoreMesh(core_axis_name="c", subcore_axis_name="s")
    nt = (nblk + _NW - 1) // _NW

    @functools.partial(
        pl.kernel, mesh=mesh,
        out_type=jax.ShapeDtypeStruct((n_pad, _H), jnp.float32),
        scratch_types=[
            pltpu.VMEM((304,), jnp.int32),
            pltpu.VMEM((_EB,), jnp.int32),
            pltpu.VMEM((_EB,), jnp.int32),
            pltpu.VMEM((_EB,), jnp.int32),
            pltpu.VMEM((_EB,), jnp.int32),
            pltpu.VMEM((_EB, _H), jnp.float32),
            pltpu.VMEM((_EB, _H), jnp.float32),
            pltpu.VMEM((_ZR, _H), jnp.float32),
            pltpu.VMEM_SHARED((16 * _ACC, _H), jnp.float32),
            pltpu.SemaphoreType.DMA,
            pltpu.SemaphoreType.DMA,
        ],
    )
    def sc_fn(g_hbm, src_hbm, ldst_hbm, bptr_hbm, s_hbm,
              bptr_v, src_v0, src_v1, ldst_v0, ldst_v1, rows_v0, rows_v1,
              zbuf, acc_sh, sem0, sem1):
        cid = lax.axis_index("c")
        sid = lax.axis_index("s")
        wid = cid * 16 + sid
        w_off = sid * _ACC
        dummy = w_off + _BD
        bufs = [(src_v0, ldst_v0, rows_v0, sem0),
                (src_v1, ldst_v1, rows_v1, sem1)]

        pltpu.sync_copy(bptr_hbm, bptr_v)

        def zero_row(i, _):
            for j in range(_H // 16):
                zbuf[i, pl.ds(j * 16, 16)] = jnp.zeros((16,), jnp.float32)
            return 0

        lax.fori_loop(0, _ZR, zero_row, 0)

        for t in range(nt):
            k = wid + _NW * t

            @pl.when(k < nblk)
            def _():
                bv = bptr_v[pl.ds(k, 16)]
                e_lo = bv[0]
                e_hi = bv[1]
                base0 = (e_lo // _EB) * _EB
                nb = (e_hi - base0 + _EB - 1) // _EB

                for r in range(_ACC // _ZR):
                    pltpu.sync_copy(
                        zbuf, acc_sh.at[pl.ds(w_off + r * _ZR, _ZR)])

                def issue(bi, buf):
                    src_v, ldst_v, rows_v, sem = buf
                    b0 = base0 + bi * _EB
                    pltpu.sync_copy(src_hbm.at[pl.ds(b0, _EB)], src_v)
                    pltpu.sync_copy(ldst_hbm.at[pl.ds(b0, _EB)], ldst_v)
                    for j in range(_EB // 16):
                        gid = b0 + j * 16 + lax.iota(jnp.int32, 16)
                        seg = ldst_v[pl.ds(j * 16, 16)]
                        ok = (gid >= e_lo) & (gid < e_hi)
                        ldst_v[pl.ds(j * 16, 16)] = jnp.where(
                            ok, seg + w_off, dummy)
                    pltpu.async_copy(g_hbm.at[src_v], rows_v, sem)

                def drain(buf):
                    src_v, ldst_v, rows_v, sem = buf
                    pltpu.make_async_copy(
                        g_hbm.at[src_v], rows_v, sem).wait()
                    pltpu.sync_copy(rows_v, acc_sh.at[ldst_v], add=True)

                @pl.when(nb > 0)
                def _():
                    issue(0, bufs[0])

                def batch_body(bi, _):
                    @pl.when(bi % 2 == 0)
                    def _():
                        issue(bi + 1, bufs[1])
                        drain(bufs[0])

                    @pl.when(bi % 2 == 1)
                    def _():
                        issue(bi + 1, bufs[0])
                        drain(bufs[1])

                    return 0

                lax.fori_loop(0, nb - 1, batch_body, 0)

                @pl.when(nb > 0)
                def _():
                    @pl.when((nb - 1) % 2 == 0)
                    def _():
                        drain(bufs[0])

                    @pl.when((nb - 1) % 2 == 1)
                    def _():
                        drain(bufs[1])

                pltpu.sync_copy(acc_sh.at[pl.ds(w_off, _BD)],
                                s_hbm.at[pl.ds(k * _BD, _BD)])

    return sc_fn(g, src_s, ldst, bptr)


def _sc_degree(ldst, bptr, n_pad, nblk):
    mesh = plsc.VectorSubcoreMesh(core_axis_name="c", subcore_axis_name="s")
    nt = (nblk + _NW - 1) // _NW

    @functools.partial(
        pl.kernel, mesh=mesh,
        out_type=jax.ShapeDtypeStruct((n_pad, _H), jnp.float32),
        scratch_types=[
            pltpu.VMEM((304,), jnp.int32),
            pltpu.VMEM((_EB,), jnp.int32),
            pltpu.VMEM((_EB, _H), jnp.float32),
            pltpu.VMEM((_ZR, _H), jnp.float32),
            pltpu.VMEM_SHARED((16 * _ACC, _H), jnp.float32),
            pltpu.SemaphoreType.DMA,
        ],
    )
    def sc_fn(ldst_hbm, bptr_hbm, deg_hbm, bptr_v, ldst_v, ones_v, zbuf,
              acc_sh, sem):
        cid = lax.axis_index("c")
        sid = lax.axis_index("s")
        wid = cid * 16 + sid
        w_off = sid * _ACC
        dummy = w_off + _BD

        pltpu.sync_copy(bptr_hbm, bptr_v)

        def fill_row(i, _):
            for j in range(_H // 16):
                ones_v[i, pl.ds(j * 16, 16)] = jnp.ones((16,), jnp.float32)
            return 0

        lax.fori_loop(0, _EB, fill_row, 0)

        def zero_row(i, _):
            for j in range(_H // 16):
                zbuf[i, pl.ds(j * 16, 16)] = jnp.zeros((16,), jnp.float32)
            return 0

        lax.fori_loop(0, _ZR, zero_row, 0)

        for t in range(nt):
            k = wid + _NW * t

            @pl.when(k < nblk)
            def _():
                bv = bptr_v[pl.ds(k, 16)]
                e_lo = bv[0]
                e_hi = bv[1]
                base0 = (e_lo // _EB) * _EB
                nb = (e_hi - base0 + _EB - 1) // _EB

                for r in range(_ACC // _ZR):
                    pltpu.sync_copy(
                        zbuf, acc_sh.at[pl.ds(w_off + r * _ZR, _ZR)])

                def batch_body(bi, _):
                    b0 = base0 + bi * _EB
                    pltpu.sync_copy(ldst_hbm.at[pl.ds(b0, _EB)], ldst_v)
                    for j in range(_EB // 16):
                        gid = b0 + j * 16 + lax.iota(jnp.int32, 16)
                        seg = ldst_v[pl.ds(j * 16, 16)]
                        ok = (gid >= e_lo) & (gid < e_hi)
                        ldst_v[pl.ds(j * 16, 16)] = jnp.where(
                            ok, seg + w_off, dummy)
                    pltpu.sync_copy(ones_v, acc_sh.at[ldst_v], add=True)
                    return 0

                lax.fori_loop(0, nb, batch_body, 0)
                pltpu.sync_copy(acc_sh.at[pl.ds(w_off, _BD)],
                                deg_hbm.at[pl.ds(k * _BD, _BD)])

    return sc_fn(ldst, bptr)


def kernel(x, edge_index, num_graphs, W0, b0, Wg, bg, gamma, beta,
           Wp1, bp1, Wp2, bp2, Wm1, bm1, Wm2, bm2):
    n = x.shape[0]
    G = n // _NPG
    G_pad = ((G + _GB - 1) // _GB) * _GB
    n_pad = G_pad * _NPG
    steps = n_pad // _R
    itp = False

    src = edge_index[0]
    dst = edge_index[1]
    src_s, ldst, bptr, nblk, _ = _sc_prep(src, dst, n_pad)
    deg = _sc_degree(ldst, bptr, n_pad, nblk)[:, :1]

    x_pad = jnp.pad(x, ((0, n_pad - n), (0, 0)))

    row_spec = pl.BlockSpec((_R, _H), lambda i: (i, 0))
    col_spec = pl.BlockSpec((_R, 1), lambda i: (i, 0))
    stat_spec = pl.BlockSpec((1, _H), lambda i: (0, 0))

    def full(shape):
        return pl.BlockSpec(shape, lambda i: tuple(0 for _ in shape))

    h, g = pl.pallas_call(
        _mm_kernel,
        grid=(steps,),
        in_specs=[row_spec, full((_H, _H)), full((1, _H)), full((_H, _H)),
                  col_spec],
        out_specs=[row_spec, row_spec],
        out_shape=[
            jax.ShapeDtypeStruct((n_pad, _H), jnp.float32),
            jax.ShapeDtypeStruct((n_pad, _H), jnp.float32),
        ],
        interpret=itp,
    )(x_pad, W0, b0[None, :], Wg[0], deg)

    for i in range(_L):
        res = h
        S = _sc_scatter(g, src_s, ldst, bptr, n_pad, nblk)

        a, s1, s2 = pl.pallas_call(
            functools.partial(_post_scatter_kernel, n_true=n, rows=_R),
            grid=(steps,),
            in_specs=[row_spec, row_spec, col_spec, full((1, _H))],
            out_specs=[row_spec, stat_spec, stat_spec],
            out_shape=[
                jax.ShapeDtypeStruct((n_pad, _H), jnp.float32),
                jax.ShapeDtypeStruct((1, _H), jnp.float32),
                jax.ShapeDtypeStruct((1, _H), jnp.float32),
            ],
            interpret=itp,
        )(S, g, deg, bg[i][None, :])

        if i < _L - 1:
            h, g = pl.pallas_call(
                functools.partial(_bn_relu_res_mm_kernel, n_true=float(n)),
                grid=(steps,),
                in_specs=[row_spec, row_spec, stat_spec, stat_spec,
                          full((1, _H)), full((1, _H)), full((_H, _H)),
                          col_spec],
                out_specs=[row_spec, row_spec],
                out_shape=[
                    jax.ShapeDtypeStruct((n_pad, _H), jnp.float32),
                    jax.ShapeDtypeStruct((n_pad, _H), jnp.float32),
                ],
                interpret=itp,
            )(a, res, s1, s2, gamma[i][None, :], beta[i][None, :],
              Wg[i + 1], deg)
        else:
            last = (a, res, s1, s2)

    U1 = np.zeros((_NBR, _NPG), np.float32)
    U1[np.arange(_NBR), _BRANCH_U] = 1.0
    V1 = np.zeros((_NBR, _NPG), np.float32)
    V1[np.arange(_NBR), _BRANCH_V] = 1.0
    BU = jnp.asarray(np.kron(np.eye(_GB, dtype=np.float32), U1))
    BV = jnp.asarray(np.kron(np.eye(_GB, dtype=np.float32), V1))
    BD = BU - BV
    EB = _GB * _NBR

    a, res, s1, s2 = last
    out = pl.pallas_call(
        functools.partial(_final_kernel, n_true=float(n)),
        grid=(G_pad // _GB,),
        in_specs=[row_spec, row_spec, stat_spec, stat_spec,
                  full((1, _H)), full((1, _H)),
                  full((EB, _R)), full((EB, _R)), full((EB, _R)),
                  full((_H, 16)), full((1, 16)), full((16, 1)), full((1, 1)),
                  full((_H, _H)), full((_H, _H)), full((1, _H)),
                  full((1, _H)), full((_H, 1)), full((1, 1))],
        out_specs=pl.BlockSpec((EB, 1), lambda i: (i, 0)),
        out_shape=jax.ShapeDtypeStruct((G_pad * _NBR, 1), jnp.float32),
        interpret=itp,
    )(a, res, s1, s2, gamma[_L - 1][None, :], beta[_L - 1][None, :],
      BU, BV, BD,
      Wp1, bp1[None, :], Wp2, bp2[None, :],
      Wm1[:_H], Wm1[_H:2 * _H], Wm1[2 * _H:2 * _H + 1],
      bm1[None, :], Wm2, bm2[None, :])

    return out[:G * _NBR]
